# baseline (speedup 1.0000x reference)
import functools

import jax
import jax.numpy as jnp
from jax import lax
from jax.experimental import pallas as pl
from jax.experimental.pallas import tpu as pltpu
from jax.experimental.pallas import tpu_sc as plsc

B = 4096
S = 200
D = 64
NC = 2
NS = 16
NW = NC * NS
ROWS_PER_W = B // NW
CHUNK = 100
NCHUNK = (2 * S) // CHUNK


def _sc_pooled_features(idx_all, emb_table):
    mesh = plsc.VectorSubcoreMesh(
        core_axis_name="c", subcore_axis_name="s", num_cores=NC, num_subcores=NS
    )

    @functools.partial(
        pl.kernel,
        out_type=jax.ShapeDtypeStruct((B, 2 * D), jnp.float32),
        mesh=mesh,
        scratch_types=[
            pltpu.VMEM((NCHUNK, CHUNK), jnp.int32),
            pltpu.VMEM((NCHUNK, CHUNK, D), jnp.float32),
            pltpu.VMEM((ROWS_PER_W, 2 * D), jnp.float32),
            pltpu.SemaphoreType.DMA,
        ],
        compiler_params=pltpu.CompilerParams(use_tc_tiling_on_sc=False),
    )
    def feat_kernel(idx_hbm, table_hbm, out_hbm, idx_v, rows_v, feat_v, sem):
        wid = lax.axis_index("s") * NC + lax.axis_index("c")
        base = wid * ROWS_PER_W

        def per_row(i, _):
            pltpu.sync_copy(idx_hbm.at[base + i], idx_v)
            copies = [
                pltpu.async_copy(table_hbm.at[idx_v.at[c]], rows_v.at[c], sem)
                for c in range(NCHUNK)
            ]
            for cp in copies:
                cp.wait()

            def pooled_max(c0, c1):
                def body(j, accs):
                    out = []
                    for k in range(4):
                        v0 = rows_v[c0, j, pl.ds(k * 16, 16)]
                        v1 = rows_v[c1, j, pl.ds(k * 16, 16)]
                        out.append(jnp.maximum(accs[k], jnp.maximum(v0, v1)))
                    return tuple(out)

                init = tuple(
                    jnp.full((16,), -jnp.inf, jnp.float32) for _ in range(4)
                )
                return lax.fori_loop(0, CHUNK, body, init)

            p_acc = pooled_max(0, 1)
            h_acc = pooled_max(2, 3)
            for k in range(4):
                feat_v[i, pl.ds(k * 16, 16)] = p_acc[k]
                feat_v[i, pl.ds(D + k * 16, 16)] = h_acc[k]
            return _

        lax.fori_loop(0, ROWS_PER_W, per_row, None)
        pltpu.sync_copy(feat_v, out_hbm.at[pl.ds(base, ROWS_PER_W)])

    return feat_kernel(idx_all, emb_table)


def _tc_head(feat, W, b):
    def head_kernel(x_ref, w_ref, b_ref, o_ref):
        z = jnp.sum(x_ref[...] * w_ref[...], axis=1, keepdims=True)
        o_ref[...] = jax.nn.sigmoid(z + b_ref[0])

    return pl.pallas_call(
        head_kernel,
        in_specs=[
            pl.BlockSpec(memory_space=pltpu.VMEM),
            pl.BlockSpec(memory_space=pltpu.VMEM),
            pl.BlockSpec(memory_space=pltpu.SMEM),
        ],
        out_shape=jax.ShapeDtypeStruct((B, 1), jnp.float32),
    )(feat, W, b)


def kernel(premise, hypothesis, emb_table, W, b):
    idx_all = jnp.concatenate(
        [premise.astype(jnp.int32), hypothesis.astype(jnp.int32)], axis=1
    ).reshape(B, NCHUNK, CHUNK)
    feat = _sc_pooled_features(idx_all, emb_table)
    return jnp.ravel(_tc_head(feat, W, b))


# speedup vs baseline: 1.2696x; 1.2696x over previous
import functools

import jax
import jax.numpy as jnp
from jax import lax
from jax.experimental import pallas as pl
from jax.experimental.pallas import tpu as pltpu
from jax.experimental.pallas import tpu_sc as plsc

B = 4096
S = 200
D = 64
NC = 2
NS = 16
NW = NC * NS
ROWS_PER_W = B // NW
CHUNK = 100
NCHUNK = (2 * S) // CHUNK


def _sc_pooled_features(idx_all, emb_table):
    mesh = plsc.VectorSubcoreMesh(
        core_axis_name="c", subcore_axis_name="s", num_cores=NC, num_subcores=NS
    )

    @functools.partial(
        pl.kernel,
        out_type=jax.ShapeDtypeStruct((B, 2 * D), jnp.float32),
        mesh=mesh,
        scratch_types=[
            pltpu.VMEM((ROWS_PER_W, NCHUNK, CHUNK), jnp.int32),
            pltpu.VMEM((2, NCHUNK, CHUNK, D), jnp.float32),
            pltpu.VMEM((ROWS_PER_W, 2 * D), jnp.float32),
            pltpu.SemaphoreType.DMA,
            pltpu.SemaphoreType.DMA,
        ],
        compiler_params=pltpu.CompilerParams(use_tc_tiling_on_sc=False),
    )
    def feat_kernel(idx_hbm, table_hbm, out_hbm, idx_v, rows_v, feat_v, s0, s1):
        wid = lax.axis_index("s") * NC + lax.axis_index("c")
        base = wid * ROWS_PER_W
        sems = (s0, s1)

        pltpu.sync_copy(idx_hbm.at[pl.ds(base, ROWS_PER_W)], idx_v)

        def fire(row, slot):
            for c in range(NCHUNK):
                pltpu.async_copy(
                    table_hbm.at[idx_v.at[row, c]], rows_v.at[slot, c], sems[slot]
                )

        def drain(slot):
            for c in range(NCHUNK):
                pltpu.make_async_copy(
                    table_hbm.at[pl.ds(0, CHUNK)], rows_v.at[slot, c], sems[slot]
                ).wait()

        def compute(row, slot):
            def pooled_max(c0, c1):
                def body(j, accs):
                    out = []
                    for k in range(4):
                        v0 = rows_v[slot, c0, j, pl.ds(k * 16, 16)]
                        v1 = rows_v[slot, c1, j, pl.ds(k * 16, 16)]
                        out.append(jnp.maximum(accs[k], jnp.maximum(v0, v1)))
                    return tuple(out)

                init = tuple(
                    jnp.full((16,), -jnp.inf, jnp.float32) for _ in range(4)
                )
                return lax.fori_loop(0, CHUNK, body, init, unroll=2)

            p_acc = pooled_max(0, 1)
            h_acc = pooled_max(2, 3)
            for k in range(4):
                feat_v[row, pl.ds(k * 16, 16)] = p_acc[k]
                feat_v[row, pl.ds(D + k * 16, 16)] = h_acc[k]

        fire(0, 0)

        def pair(g, _):
            row = 2 * g
            fire(row + 1, 1)
            drain(0)
            compute(row, 0)

            @pl.when(row + 2 < ROWS_PER_W)
            def _fire_next():
                fire(row + 2, 0)

            drain(1)
            compute(row + 1, 1)
            return _

        lax.fori_loop(0, ROWS_PER_W // 2, pair, None)
        pltpu.sync_copy(feat_v, out_hbm.at[pl.ds(base, ROWS_PER_W)])

    return feat_kernel(idx_all, emb_table)


def _tc_head(feat, W, b):
    def head_kernel(x_ref, w_ref, b_ref, o_ref):
        z = jnp.sum(x_ref[...] * w_ref[...], axis=1, keepdims=True)
        o_ref[...] = jax.nn.sigmoid(z + b_ref[0])

    return pl.pallas_call(
        head_kernel,
        in_specs=[
            pl.BlockSpec(memory_space=pltpu.VMEM),
            pl.BlockSpec(memory_space=pltpu.VMEM),
            pl.BlockSpec(memory_space=pltpu.SMEM),
        ],
        out_shape=jax.ShapeDtypeStruct((B, 1), jnp.float32),
    )(feat, W, b)


def kernel(premise, hypothesis, emb_table, W, b):
    idx_all = jnp.concatenate(
        [premise.astype(jnp.int32), hypothesis.astype(jnp.int32)], axis=1
    ).reshape(B, NCHUNK, CHUNK)
    feat = _sc_pooled_features(idx_all, emb_table)
    return jnp.ravel(_tc_head(feat, W, b))
